# baseline (speedup 1.0000x reference)
import functools

import jax
import jax.numpy as jnp
from jax import lax
from jax.experimental import pallas as pl
from jax.experimental.pallas import tpu as pltpu
from jax.experimental.pallas import tpu_sc as plsc

C = 384
HMAP = 128
WMAP = 128
HW = HMAP * WMAP
KTOP = 2048
MAXF = 4096
NLVL = 3
NEG_INF = float("-inf")


def _k0_body(x0_ref, x1_ref, x2_ref, x3_ref, out_ref):
    def sum_i(r):
        return ((r[:, :, 0, :] + r[:, :, 1, :]) + r[:, :, 2, :]) + r[:, :, 3, :]

    s0 = sum_i(x0_ref[...])
    s1 = sum_i(x1_ref[...])
    s2 = sum_i(x2_ref[...])
    s3 = sum_i(x3_ref[...])
    t0 = s0 + s2
    t1 = s1 + s3
    out_ref[...] = (t0 + t1) * (1.0 / 16.0)


def _downsample(img):
    views = [img[:, :, j::4].reshape(3, HMAP, 4, WMAP) for j in range(4)]
    return pl.pallas_call(
        _k0_body,
        out_shape=jax.ShapeDtypeStruct((3, HMAP, WMAP), jnp.float32),
    )(*views)


def _k1a_body(x_ref, w_ref, v_ref, score_ref):
    X = x_ref[...]
    Wm = w_ref[0]
    vv = v_ref[0]
    Xb = X.astype(jnp.bfloat16)
    Wb = Wm.astype(jnp.bfloat16)
    F = jax.lax.dot_general(Wb, Xb, (((0,), (0,)), ((), ())),
                            preferred_element_type=jnp.float32)
    F = jnp.maximum(F, 0.0)
    Fb = F.astype(jnp.bfloat16)
    vb = vv.astype(jnp.bfloat16)
    lg = jax.lax.dot_general(vb, Fb, (((1,), (0,)), ((), ())),
                             preferred_element_type=jnp.float32)
    score_ref[0] = jax.nn.softplus(lg)


def _score_maps(X, Wcat, vcat):
    return pl.pallas_call(
        _k1a_body,
        grid=(NLVL,),
        in_specs=[
            pl.BlockSpec((3, HW), lambda l: (0, 0)),
            pl.BlockSpec((1, 3, C), lambda l: (l, 0, 0)),
            pl.BlockSpec((1, 1, C), lambda l: (l, 0, 0)),
        ],
        out_specs=pl.BlockSpec((1, 1, HW), lambda l: (l, 0, 0)),
        out_shape=jax.ShapeDtypeStruct((NLVL, 1, HW), jnp.float32),
    )(X, Wcat, vcat)


def _k1b_body(score_ref, nms_ref, di_ref, dj_ref):
    s = score_ref[0]
    negr = jnp.full((1, WMAP), NEG_INF, jnp.float32)
    negc = jnp.full((HMAP, 1), NEG_INF, jnp.float32)
    s_dn = jnp.concatenate([s[1:, :], negr], axis=0)
    s_up = jnp.concatenate([negr, s[:-1, :]], axis=0)
    mv = jnp.maximum(jnp.maximum(s, s_dn), s_up)
    m_r = jnp.concatenate([mv[:, 1:], negc], axis=1)
    m_l = jnp.concatenate([negc, mv[:, :-1]], axis=1)
    lmax = jnp.maximum(jnp.maximum(mv, m_r), m_l)
    nms_ref[0] = jnp.where(lmax == s, s, 0.0)

    spr = jnp.concatenate([s[0:1, :], s, s[HMAP - 1:HMAP, :]], axis=0)
    sp = jnp.concatenate([spr[:, 0:1], spr, spr[:, WMAP - 1:WMAP]], axis=1)
    di = 0.5 * (sp[2:, 1:-1] - sp[:-2, 1:-1])
    dj = 0.5 * (sp[1:-1, 2:] - sp[1:-1, :-2])
    dii = sp[2:, 1:-1] - 2.0 * s + sp[:-2, 1:-1]
    djj = sp[1:-1, 2:] - 2.0 * s + sp[1:-1, :-2]
    dij = 0.25 * (sp[2:, 2:] - sp[2:, :-2] - sp[:-2, 2:] + sp[:-2, :-2])
    det = dii * djj - dij * dij
    safe = jnp.abs(det) > 1e-6
    dets = jnp.where(safe, det, 1.0)
    di_ref[0] = jnp.where(safe, -(djj * di - dij * dj) / dets, 2.0)
    dj_ref[0] = jnp.where(safe, -(dii * dj - dij * di) / dets, 2.0)


def _nms_disp(score3):
    outs = (jax.ShapeDtypeStruct((NLVL, HMAP, WMAP), jnp.float32),) * 3
    return pl.pallas_call(
        _k1b_body,
        grid=(NLVL,),
        in_specs=[pl.BlockSpec((1, HMAP, WMAP), lambda l: (l, 0, 0))],
        out_specs=(pl.BlockSpec((1, HMAP, WMAP), lambda l: (l, 0, 0)),) * 3,
        out_shape=outs,
    )(score3)


P_CAP = 8192
SORTN = P_CAP + 80
NCAND = NLVL * KTOP
NW = 16


def _iota16():
    return lax.iota(jnp.int32, 16)


def _radix_pass(kin, iin, kout, iout, hist_v, offs_v, T, shift):
    def zbody(k, _):
        hist_v[pl.ds(k * 16, 16)] = jnp.zeros((16,), jnp.int32)
        return 0
    lax.fori_loop(0, 32, zbody, 0)

    def hbody(tt, _):
        for u in range(4):
            li = _iota16() * T + (tt * 4 + u)
            kv = plsc.load_gather(kin, [li])
            d = 31 - (lax.shift_right_logical(kv, shift) & 31)
            slot = d * 16 + _iota16()
            plsc.addupdate_scatter(hist_v, [slot], jnp.ones((16,), jnp.int32))
        return 0
    lax.fori_loop(0, T // 4, hbody, 0)

    def sbody(k, carry):
        v = hist_v[pl.ds(k * 16, 16)]
        cs = plsc.cumsum(v)
        offs_v[pl.ds(k * 16, 16)] = (cs - v) + carry
        return carry + jnp.sum(v)
    lax.fori_loop(0, 32, sbody, jnp.int32(0))

    def pbody(tt, _):
        for u in range(4):
            li = _iota16() * T + (tt * 4 + u)
            kv = plsc.load_gather(kin, [li])
            pv = plsc.load_gather(iin, [li])
            d = 31 - (lax.shift_right_logical(kv, shift) & 31)
            slot = d * 16 + _iota16()
            off = plsc.load_gather(offs_v, [slot])
            plsc.store_scatter(kout, [off], kv)
            plsc.store_scatter(iout, [off], pv)
            plsc.store_scatter(offs_v, [slot], off + 1)
        return 0
    lax.fori_loop(0, T // 4, pbody, 0)


def _k2_body(snms, dispi, dispj, xds,
             scores_o, kpflat_o, xcflat_o, wts_o, lvl_o, msk_o,
             map_v, disp_v,
             skey_a, sidx_a, skey_b, sidx_b,
             zidx_v, hist_v, offs_v,
             cidx_v, cval_v, cdi_v, cdj_v,
             f_sc, f_kx, f_ky, f_vd, f_wi, f_wj, f_bs,
             spv_v, b_loc, s_loc, kp_loc, w_loc, wts4_loc, xc4_loc,
             sp_fields, sp_sortedp):
    wid = lax.axis_index("s")

    @pl.when(wid < NLVL)
    def _phase_a():
        l = wid
        pltpu.sync_copy(snms.at[pl.ds(l * HW, HW)], map_v)

        def comp_body(tt, cnt):
            for u in range(4):
                t = tt * 4 + u
                v = map_v[pl.ds(t * 16, 16)]
                m = v > 0.0
                csm = plsc.cumsum(m.astype(jnp.int32))
                pos = jnp.minimum(cnt + csm - 1, P_CAP - 1)
                plsc.store_scatter(skey_a, [pos], plsc.bitcast(v, jnp.int32),
                                   mask=m)
                plsc.store_scatter(sidx_a, [pos], _iota16() + t * 16, mask=m)
                cnt = cnt + jnp.sum(m.astype(jnp.int32))
            return cnt
        P = lax.fori_loop(0, HW // 64, comp_body, jnp.int32(0))
        P = jnp.minimum(P, P_CAP)
        P16 = (P + 15) // 16 * 16
        for q in range(4):
            skey_a[pl.ds(P16 + q * 16, 16)] = jnp.zeros((16,), jnp.int32)
            sidx_a[pl.ds(P16 + q * 16, 16)] = jnp.full((16,), HW, jnp.int32)
        skey_a[pl.ds(P, 16)] = jnp.zeros((16,), jnp.int32)
        sidx_a[pl.ds(P, 16)] = jnp.full((16,), HW, jnp.int32)
        T = (P + 63) // 64 * 4

        bufs = [(skey_a, sidx_a, skey_b, sidx_b),
                (skey_b, sidx_b, skey_a, sidx_a)]
        for pno in range(7):
            kin, iin, kout, iout = bufs[pno % 2]
            _radix_pass(kin, iin, kout, iout, hist_v, offs_v, T, pno * 5)

        zneed = KTOP - jnp.minimum(P, KTOP)

        def zbody(tt, zc):
            for u in range(4):
                t = tt * 4 + u
                v = map_v[pl.ds(t * 16, 16)]
                m = (v <= 0.0) & (zc < zneed)
                csm = plsc.cumsum(m.astype(jnp.int32))
                pos = jnp.minimum(zc + csm - 1, KTOP + 15)
                plsc.store_scatter(zidx_v, [pos], _iota16() + t * 16, mask=m)
                zc = zc + jnp.sum(m.astype(jnp.int32))
            return zc
        lax.fori_loop(0, HW // 64, zbody, jnp.int32(0))

        def cb1(r, _):
            rv = _iota16() + r * 16
            isel = rv < P
            spos = jnp.where(isel, rv, 0)
            kb = plsc.load_gather(skey_b, [spos])
            si = plsc.load_gather(sidx_b, [spos])
            zpos = jnp.where(isel, 0, jnp.minimum(rv - P, KTOP + 15))
            zi = plsc.load_gather(zidx_v, [zpos])
            idx = jnp.where(isel, si, zi)
            valb = jnp.where(isel, kb, 0)
            cidx_v[pl.ds(r * 16, 16)] = idx
            cval_v[pl.ds(r * 16, 16)] = plsc.bitcast(valb, jnp.float32)
            return 0
        lax.fori_loop(0, KTOP // 16, cb1, 0)

        pltpu.sync_copy(dispi.at[pl.ds(l * HW, HW)], disp_v)

        def cb2(r, _):
            idx = cidx_v[pl.ds(r * 16, 16)]
            cdi_v[pl.ds(r * 16, 16)] = plsc.load_gather(disp_v, [idx])
            return 0
        lax.fori_loop(0, KTOP // 16, cb2, 0)

        pltpu.sync_copy(dispj.at[pl.ds(l * HW, HW)], disp_v)

        def cb3(r, _):
            idx = cidx_v[pl.ds(r * 16, 16)]
            cdj_v[pl.ds(r * 16, 16)] = plsc.load_gather(disp_v, [idx])
            return 0
        lax.fori_loop(0, KTOP // 16, cb3, 0)

        def cb4(r, _):
            sl = pl.ds(r * 16, 16)
            idx = cidx_v[sl]
            val = cval_v[sl]
            di = cdi_v[sl]
            dj = cdj_v[sl]
            hi = lax.shift_right_logical(idx, 7)
            wi = idx & 127
            kpi = hi.astype(jnp.float32) + di
            kpj = wi.astype(jnp.float32) + dj
            valid_d = (jnp.abs(di) < 0.5) & (jnp.abs(dj) < 0.5)
            valid_b = ((kpi >= 0.0) & (kpi <= HMAP - 1.0)
                       & (kpj >= 0.0) & (kpj <= WMAP - 1.0))
            valid = valid_d & valid_b & (val > 0.0)
            vf = valid.astype(jnp.float32)
            i0 = jnp.minimum(jnp.maximum(kpi.astype(jnp.int32), 0), HMAP - 2)
            j0 = jnp.minimum(jnp.maximum(kpj.astype(jnp.int32), 0), WMAP - 2)
            f_sc[sl] = val * vf
            f_kx[sl] = kpj * 16.0 + 7.5
            f_ky[sl] = kpi * 16.0 + 7.5
            f_vd[sl] = vf
            f_wi[sl] = kpi - i0.astype(jnp.float32)
            f_wj[sl] = kpj - j0.astype(jnp.float32)
            f_bs[sl] = (i0 * WMAP + j0).astype(jnp.float32)
            return 0
        lax.fori_loop(0, KTOP // 16, cb4, 0)

        for fi, fb in enumerate((f_sc, f_kx, f_ky, f_vd, f_wi, f_wj, f_bs)):
            pltpu.sync_copy(fb, sp_fields.at[pl.ds(fi * NCAND + l * KTOP, KTOP)])

    plsc.subcore_barrier()

    @pl.when(wid == 0)
    def _phase_b():
        pltpu.sync_copy(sp_fields.at[pl.ds(0, NCAND)], map_v.at[pl.ds(0, NCAND)])

        def pb(t, _):
            sl = pl.ds(t * 16, 16)
            skey_a[sl] = plsc.bitcast(map_v[sl], jnp.int32)
            sidx_a[sl] = _iota16() + t * 16
            return 0
        lax.fori_loop(0, NCAND // 16, pb, 0)
        for pno in range(7):
            kin, iin, kout, iout = [(skey_a, sidx_a, skey_b, sidx_b),
                                    (skey_b, sidx_b, skey_a, sidx_a)][pno % 2]
            _radix_pass(kin, iin, kout, iout, hist_v, offs_v, NCAND // 16,
                        pno * 5)
        pltpu.sync_copy(sidx_b.at[pl.ds(0, MAXF)], sp_sortedp)

    plsc.subcore_barrier()

    K = MAXF // NW
    base = wid * K
    pltpu.sync_copy(sp_sortedp.at[pl.ds(base, K)], spv_v)

    def _gather_field(fi, dst_loc):
        pltpu.sync_copy(sp_fields.at[pl.ds(fi * NCAND, NCAND)], map_v.at[pl.ds(0, NCAND)])

        def gb(i, _):
            p = spv_v[pl.ds(i * 16, 16)]
            dst_loc[pl.ds(i * 16, 16)] = plsc.load_gather(map_v, [p])
            return 0
        lax.fori_loop(0, K // 16, gb, 0)

    _gather_field(0, s_loc)
    pltpu.sync_copy(s_loc, scores_o.at[pl.ds(base, K)])
    _gather_field(1, w_loc)

    def kx_b(i, _):
        sl = pl.ds(i * 16, 16)
        plsc.store_scatter(kp_loc, [(_iota16() + i * 16) * 2], w_loc[sl])
        return 0
    lax.fori_loop(0, K // 16, kx_b, 0)
    _gather_field(2, w_loc)

    def ky_b(i, _):
        sl = pl.ds(i * 16, 16)
        plsc.store_scatter(kp_loc, [(_iota16() + i * 16) * 2 + 1], w_loc[sl])
        return 0
    lax.fori_loop(0, K // 16, ky_b, 0)
    pltpu.sync_copy(kp_loc, kpflat_o.at[pl.ds(base * 2, K * 2)])
    _gather_field(3, s_loc)
    pltpu.sync_copy(s_loc, msk_o.at[pl.ds(base, K)])
    for lv in range(NLVL):
        def lv_b(i, _):
            p = spv_v[pl.ds(i * 16, 16)]
            lvl_i = lax.shift_right_logical(p, 11)
            s_loc[pl.ds(i * 16, 16)] = (lvl_i == lv).astype(jnp.float32)
            return 0
        lax.fori_loop(0, K // 16, lv_b, 0)
        pltpu.sync_copy(s_loc, lvl_o.at[pl.ds(lv * MAXF + base, K)])
    _gather_field(4, w_loc)

    def wi_b(i, _):
        s_loc[pl.ds(i * 16, 16)] = w_loc[pl.ds(i * 16, 16)]
        return 0
    lax.fori_loop(0, K // 16, wi_b, 0)
    _gather_field(5, w_loc)

    def wt_b(i, _):
        sl = pl.ds(i * 16, 16)
        wif = s_loc[sl]
        wjf = w_loc[sl]
        wts4_loc[pl.ds(0 * K + i * 16, 16)] = (1.0 - wif) * (1.0 - wjf)
        wts4_loc[pl.ds(1 * K + i * 16, 16)] = (1.0 - wif) * wjf
        wts4_loc[pl.ds(2 * K + i * 16, 16)] = wif * (1.0 - wjf)
        wts4_loc[pl.ds(3 * K + i * 16, 16)] = wif * wjf
        return 0
    lax.fori_loop(0, K // 16, wt_b, 0)
    for c in range(4):
        pltpu.sync_copy(wts4_loc.at[pl.ds(c * K, K)],
                        wts_o.at[pl.ds(c * MAXF + base, K)])
    _gather_field(6, w_loc)

    def bs_b(i, _):
        b_loc[pl.ds(i * 16, 16)] = w_loc[pl.ds(i * 16, 16)].astype(jnp.int32)
        return 0
    lax.fori_loop(0, K // 16, bs_b, 0)
    for ch in range(3):
        pltpu.sync_copy(xds.at[pl.ds(ch * HW, HW)], map_v)
        for ci, off in enumerate((0, 1, WMAP, WMAP + 1)):
            def xc_b(i, _, ci=ci, off=off, ch=ch):
                b = b_loc[pl.ds(i * 16, 16)]
                g = plsc.load_gather(map_v, [b + off])
                pos = ci * (K * 3) + (_iota16() + i * 16) * 3 + ch
                plsc.store_scatter(xc4_loc, [pos], g)
                return 0
            lax.fori_loop(0, K // 16, xc_b, 0)
    for c in range(4):
        pltpu.sync_copy(xc4_loc.at[pl.ds(c * (K * 3), K * 3)],
                        xcflat_o.at[pl.ds(c * MAXF * 3 + base * 3, K * 3)])


def _select_order(snms3, dispi3, dispj3, X):
    mesh = plsc.VectorSubcoreMesh(core_axis_name="c", subcore_axis_name="s",
                                  num_cores=1)
    f32 = jnp.float32
    run = pl.kernel(
        _k2_body,
        out_type=(
            jax.ShapeDtypeStruct((MAXF,), f32),
            jax.ShapeDtypeStruct((MAXF * 2,), f32),
            jax.ShapeDtypeStruct((4 * MAXF * 3,), f32),
            jax.ShapeDtypeStruct((4 * MAXF,), f32),
            jax.ShapeDtypeStruct((NLVL * MAXF,), f32),
            jax.ShapeDtypeStruct((MAXF,), f32),
        ),
        mesh=mesh,
        compiler_params=pltpu.CompilerParams(needs_layout_passes=False),
        scratch_types=[
            pltpu.VMEM((HW,), f32),
            pltpu.VMEM((HW,), f32),
            pltpu.VMEM((SORTN,), jnp.int32),
            pltpu.VMEM((SORTN,), jnp.int32),
            pltpu.VMEM((SORTN,), jnp.int32),
            pltpu.VMEM((SORTN,), jnp.int32),
            pltpu.VMEM((KTOP + 16,), jnp.int32),
            pltpu.VMEM((512,), jnp.int32),
            pltpu.VMEM((512,), jnp.int32),
            pltpu.VMEM((KTOP,), jnp.int32),
            pltpu.VMEM((KTOP,), f32),
            pltpu.VMEM((KTOP,), f32),
            pltpu.VMEM((KTOP,), f32),
            pltpu.VMEM((KTOP,), f32),
            pltpu.VMEM((KTOP,), f32),
            pltpu.VMEM((KTOP,), f32),
            pltpu.VMEM((KTOP,), f32),
            pltpu.VMEM((KTOP,), f32),
            pltpu.VMEM((KTOP,), f32),
            pltpu.VMEM((KTOP,), f32),
            pltpu.VMEM((MAXF // NW,), jnp.int32),
            pltpu.VMEM((MAXF // NW,), jnp.int32),
            pltpu.VMEM((MAXF // NW,), f32),
            pltpu.VMEM((MAXF // NW * 2,), f32),
            pltpu.VMEM((MAXF // NW,), f32),
            pltpu.VMEM((MAXF // NW * 4,), f32),
            pltpu.VMEM((MAXF // NW * 3 * 4,), f32),
            pltpu.VMEM_SHARED((7 * NCAND,), f32),
            pltpu.VMEM_SHARED((MAXF,), jnp.int32),
        ],
    )
    return run(snms3, dispi3, dispj3, X)


def _k3_body(xc_ref, wts_ref, lvl_ref, msk_ref, w_ref, out_ref):
    acc = jnp.zeros((out_ref.shape[1], C), jnp.float32)
    for l in range(NLVL):
        Wb = w_ref[l].astype(jnp.bfloat16)
        lacc = jnp.zeros((out_ref.shape[1], C), jnp.float32)
        for c in range(4):
            A = xc_ref[c]
            Fb = jax.lax.dot_general(A.astype(jnp.bfloat16), Wb,
                                     (((1,), (0,)), ((), ())),
                                     preferred_element_type=jnp.float32)
            Fb = jnp.maximum(Fb, 0.0)
            lacc = lacc + wts_ref[c][:, None] * Fb
        acc = acc + lvl_ref[l][:, None] * lacc
    desc = acc * msk_ref[0][:, None]
    nrm = jnp.sqrt(jnp.sum(desc * desc, axis=1, keepdims=True))
    out_ref[0] = desc / (nrm + 1e-8)


def _descriptors(xc, wts, lvl1h, vmask, Wcat):
    R = 512
    return pl.pallas_call(
        _k3_body,
        grid=(MAXF // R,),
        in_specs=[
            pl.BlockSpec((4, R, 3), lambda b: (0, b, 0)),
            pl.BlockSpec((4, R), lambda b: (0, b)),
            pl.BlockSpec((NLVL, R), lambda b: (0, b)),
            pl.BlockSpec((1, R), lambda b: (0, b)),
            pl.BlockSpec((NLVL, 3, C), lambda b: (0, 0, 0)),
        ],
        out_specs=pl.BlockSpec((1, R, C), lambda b: (b, 0, 0)),
        out_shape=jax.ShapeDtypeStruct((MAXF // R, R, C), jnp.float32),
    )(xc, wts, lvl1h, vmask, Wcat)


def kernel(images, W_early, W_middle, W_deep, v_early, v_middle, v_deep):
    img = images[0]
    x_ds = _downsample(img)
    X = x_ds.reshape(3, HW)
    Wcat = jnp.stack([W_early, W_middle, W_deep], axis=0)
    vcat = jnp.stack([v_early, v_middle, v_deep], axis=0).reshape(NLVL, 1, C)

    score3 = _score_maps(X, Wcat, vcat).reshape(NLVL, HMAP, WMAP)
    s_nms, disp_i, disp_j = _nms_disp(score3)

    scores_out, kpflat, xcflat, wtsf, lvlf, mskflat = _select_order(
        s_nms.reshape(NLVL * HW), disp_i.reshape(NLVL * HW),
        disp_j.reshape(NLVL * HW), X.reshape(NLVL * HW))
    wts = wtsf.reshape(4, MAXF)
    lvl1h = lvlf.reshape(NLVL, MAXF)

    keypoints = kpflat.reshape(MAXF, 2)
    xc = xcflat.reshape(4, MAXF, 3)
    vmask = mskflat.reshape(1, MAXF)
    descriptors = _descriptors(xc, wts, lvl1h, vmask, Wcat).reshape(MAXF, C)
    return keypoints, descriptors, scores_out


# speedup vs baseline: 1.0439x; 1.0439x over previous
import functools

import jax
import jax.numpy as jnp
from jax import lax
from jax.experimental import pallas as pl
from jax.experimental.pallas import tpu as pltpu
from jax.experimental.pallas import tpu_sc as plsc

C = 384
HMAP = 128
WMAP = 128
HW = HMAP * WMAP
KTOP = 2048
MAXF = 4096
NLVL = 3
NEG_INF = float("-inf")


def _k0_body(x0_ref, x1_ref, x2_ref, x3_ref, out_ref):
    def sum_i(r):
        return ((r[:, :, 0, :] + r[:, :, 1, :]) + r[:, :, 2, :]) + r[:, :, 3, :]

    s0 = sum_i(x0_ref[...])
    s1 = sum_i(x1_ref[...])
    s2 = sum_i(x2_ref[...])
    s3 = sum_i(x3_ref[...])
    t0 = s0 + s2
    t1 = s1 + s3
    out_ref[...] = (t0 + t1) * (1.0 / 16.0)


def _downsample(img):
    views = [img[:, :, j::4].reshape(3, HMAP, 4, WMAP) for j in range(4)]
    return pl.pallas_call(
        _k0_body,
        out_shape=jax.ShapeDtypeStruct((3, HMAP, WMAP), jnp.float32),
    )(*views)


def _k1a_body(x_ref, w_ref, v_ref, score_ref):
    X = x_ref[...]
    Wm = w_ref[0]
    vv = v_ref[0]
    Xb = X.astype(jnp.bfloat16)
    Wb = Wm.astype(jnp.bfloat16)
    F = jax.lax.dot_general(Wb, Xb, (((0,), (0,)), ((), ())),
                            preferred_element_type=jnp.float32)
    F = jnp.maximum(F, 0.0)
    Fb = F.astype(jnp.bfloat16)
    vb = vv.astype(jnp.bfloat16)
    lg = jax.lax.dot_general(vb, Fb, (((1,), (0,)), ((), ())),
                             preferred_element_type=jnp.float32)
    score_ref[0] = jax.nn.softplus(lg)


def _score_maps(X, Wcat, vcat):
    return pl.pallas_call(
        _k1a_body,
        grid=(NLVL,),
        in_specs=[
            pl.BlockSpec((3, HW), lambda l: (0, 0)),
            pl.BlockSpec((1, 3, C), lambda l: (l, 0, 0)),
            pl.BlockSpec((1, 1, C), lambda l: (l, 0, 0)),
        ],
        out_specs=pl.BlockSpec((1, 1, HW), lambda l: (l, 0, 0)),
        out_shape=jax.ShapeDtypeStruct((NLVL, 1, HW), jnp.float32),
    )(X, Wcat, vcat)


def _k1b_body(score_ref, nms_ref, di_ref, dj_ref):
    s = score_ref[0]
    negr = jnp.full((1, WMAP), NEG_INF, jnp.float32)
    negc = jnp.full((HMAP, 1), NEG_INF, jnp.float32)
    s_dn = jnp.concatenate([s[1:, :], negr], axis=0)
    s_up = jnp.concatenate([negr, s[:-1, :]], axis=0)
    mv = jnp.maximum(jnp.maximum(s, s_dn), s_up)
    m_r = jnp.concatenate([mv[:, 1:], negc], axis=1)
    m_l = jnp.concatenate([negc, mv[:, :-1]], axis=1)
    lmax = jnp.maximum(jnp.maximum(mv, m_r), m_l)
    nms_ref[0] = jnp.where(lmax == s, s, 0.0)

    spr = jnp.concatenate([s[0:1, :], s, s[HMAP - 1:HMAP, :]], axis=0)
    sp = jnp.concatenate([spr[:, 0:1], spr, spr[:, WMAP - 1:WMAP]], axis=1)
    di = 0.5 * (sp[2:, 1:-1] - sp[:-2, 1:-1])
    dj = 0.5 * (sp[1:-1, 2:] - sp[1:-1, :-2])
    dii = sp[2:, 1:-1] - 2.0 * s + sp[:-2, 1:-1]
    djj = sp[1:-1, 2:] - 2.0 * s + sp[1:-1, :-2]
    dij = 0.25 * (sp[2:, 2:] - sp[2:, :-2] - sp[:-2, 2:] + sp[:-2, :-2])
    det = dii * djj - dij * dij
    safe = jnp.abs(det) > 1e-6
    dets = jnp.where(safe, det, 1.0)
    di_ref[0] = jnp.where(safe, -(djj * di - dij * dj) / dets, 2.0)
    dj_ref[0] = jnp.where(safe, -(dii * dj - dij * di) / dets, 2.0)


def _nms_disp(score3):
    outs = (jax.ShapeDtypeStruct((NLVL, HMAP, WMAP), jnp.float32),) * 3
    return pl.pallas_call(
        _k1b_body,
        grid=(NLVL,),
        in_specs=[pl.BlockSpec((1, HMAP, WMAP), lambda l: (l, 0, 0))],
        out_specs=(pl.BlockSpec((1, HMAP, WMAP), lambda l: (l, 0, 0)),) * 3,
        out_shape=outs,
    )(score3)


P_CAP = 8192
SORTN = P_CAP + 80
NCAND = NLVL * KTOP
NW = 16


def _iota16():
    return lax.iota(jnp.int32, 16)


def _radix_pass(kin, iin, kout, iout, hist_v, offs_v, T, shift):
    def zbody(k, _):
        hist_v[pl.ds(k * 16, 16)] = jnp.zeros((16,), jnp.int32)
        return 0
    lax.fori_loop(0, 32, zbody, 0)

    def hbody(tt, _):
        for u in range(4):
            li = _iota16() * T + (tt * 4 + u)
            kv = plsc.load_gather(kin, [li])
            d = 31 - (lax.shift_right_logical(kv, shift) & 31)
            slot = d * 16 + _iota16()
            plsc.addupdate_scatter(hist_v, [slot], jnp.ones((16,), jnp.int32))
        return 0
    lax.fori_loop(0, T // 4, hbody, 0)

    def sbody(k, carry):
        v = hist_v[pl.ds(k * 16, 16)]
        cs = plsc.cumsum(v)
        offs_v[pl.ds(k * 16, 16)] = (cs - v) + carry
        return carry + jnp.sum(v)
    lax.fori_loop(0, 32, sbody, jnp.int32(0))

    def pbody(tt, _):
        for u in range(4):
            li = _iota16() * T + (tt * 4 + u)
            kv = plsc.load_gather(kin, [li])
            pv = plsc.load_gather(iin, [li])
            d = 31 - (lax.shift_right_logical(kv, shift) & 31)
            slot = d * 16 + _iota16()
            off = plsc.load_gather(offs_v, [slot])
            plsc.store_scatter(kout, [off], kv)
            plsc.store_scatter(iout, [off], pv)
            plsc.store_scatter(offs_v, [slot], off + 1)
        return 0
    lax.fori_loop(0, T // 4, pbody, 0)


def _k2_body(snms, dispi, dispj, xds,
             scores_o, kpflat_o, xcflat_o, wts_o, lvl_o, msk_o,
             map_v, disp_v,
             skey_a, sidx_a, skey_b, sidx_b,
             zidx_v, hist_v, offs_v,
             cidx_v, cval_v, cdi_v, cdj_v,
             f_sc, f_kx, f_ky, f_vd, f_wi, f_wj, f_bs,
             spv_v, b_loc, s_loc, kp_loc, w_loc, wts4_loc, xc4_loc,
             sp_fields, sp_sortedp):
    wid = lax.axis_index("s")

    @pl.when(wid < NLVL)
    def _phase_a():
        l = wid
        pltpu.sync_copy(snms.at[pl.ds(l * HW, HW)], map_v)

        def comp_body(tt, cnt):
            for u in range(4):
                t = tt * 4 + u
                v = map_v[pl.ds(t * 16, 16)]
                m = v > 0.0
                csm = plsc.cumsum(m.astype(jnp.int32))
                pos = jnp.minimum(cnt + csm - 1, P_CAP - 1)
                plsc.store_scatter(skey_a, [pos], plsc.bitcast(v, jnp.int32),
                                   mask=m)
                plsc.store_scatter(sidx_a, [pos], _iota16() + t * 16, mask=m)
                cnt = cnt + jnp.sum(m.astype(jnp.int32))
            return cnt
        P = lax.fori_loop(0, HW // 64, comp_body, jnp.int32(0))
        P = jnp.minimum(P, P_CAP)
        P16 = (P + 15) // 16 * 16
        for q in range(4):
            skey_a[pl.ds(P16 + q * 16, 16)] = jnp.zeros((16,), jnp.int32)
            sidx_a[pl.ds(P16 + q * 16, 16)] = jnp.full((16,), HW, jnp.int32)
        skey_a[pl.ds(P, 16)] = jnp.zeros((16,), jnp.int32)
        sidx_a[pl.ds(P, 16)] = jnp.full((16,), HW, jnp.int32)
        T = (P + 63) // 64 * 4

        bufs = [(skey_a, sidx_a, skey_b, sidx_b),
                (skey_b, sidx_b, skey_a, sidx_a)]
        for pno in range(7):
            kin, iin, kout, iout = bufs[pno % 2]
            _radix_pass(kin, iin, kout, iout, hist_v, offs_v, T, pno * 5)

        zneed = KTOP - jnp.minimum(P, KTOP)

        def zbody(tt, zc):
            for u in range(4):
                t = tt * 4 + u
                v = map_v[pl.ds(t * 16, 16)]
                m = (v <= 0.0) & (zc < zneed)
                csm = plsc.cumsum(m.astype(jnp.int32))
                pos = jnp.minimum(zc + csm - 1, KTOP + 15)
                plsc.store_scatter(zidx_v, [pos], _iota16() + t * 16, mask=m)
                zc = zc + jnp.sum(m.astype(jnp.int32))
            return zc
        lax.fori_loop(0, HW // 64, zbody, jnp.int32(0))

        def cb1(r, _):
            rv = _iota16() + r * 16
            isel = rv < P
            spos = jnp.where(isel, rv, 0)
            kb = plsc.load_gather(skey_b, [spos])
            si = plsc.load_gather(sidx_b, [spos])
            zpos = jnp.where(isel, 0, jnp.minimum(rv - P, KTOP + 15))
            zi = plsc.load_gather(zidx_v, [zpos])
            idx = jnp.where(isel, si, zi)
            valb = jnp.where(isel, kb, 0)
            cidx_v[pl.ds(r * 16, 16)] = idx
            cval_v[pl.ds(r * 16, 16)] = plsc.bitcast(valb, jnp.float32)
            return 0
        lax.fori_loop(0, KTOP // 16, cb1, 0)

        pltpu.sync_copy(dispi.at[pl.ds(l * HW, HW)], disp_v)

        def cb2(r, _):
            idx = cidx_v[pl.ds(r * 16, 16)]
            cdi_v[pl.ds(r * 16, 16)] = plsc.load_gather(disp_v, [idx])
            return 0
        lax.fori_loop(0, KTOP // 16, cb2, 0)

        pltpu.sync_copy(dispj.at[pl.ds(l * HW, HW)], disp_v)

        def cb3(r, _):
            idx = cidx_v[pl.ds(r * 16, 16)]
            cdj_v[pl.ds(r * 16, 16)] = plsc.load_gather(disp_v, [idx])
            return 0
        lax.fori_loop(0, KTOP // 16, cb3, 0)

        def cb4(r, _):
            sl = pl.ds(r * 16, 16)
            idx = cidx_v[sl]
            val = cval_v[sl]
            di = cdi_v[sl]
            dj = cdj_v[sl]
            hi = lax.shift_right_logical(idx, 7)
            wi = idx & 127
            kpi = hi.astype(jnp.float32) + di
            kpj = wi.astype(jnp.float32) + dj
            valid_d = (jnp.abs(di) < 0.5) & (jnp.abs(dj) < 0.5)
            valid_b = ((kpi >= 0.0) & (kpi <= HMAP - 1.0)
                       & (kpj >= 0.0) & (kpj <= WMAP - 1.0))
            valid = valid_d & valid_b & (val > 0.0)
            vf = valid.astype(jnp.float32)
            i0 = jnp.minimum(jnp.maximum(kpi.astype(jnp.int32), 0), HMAP - 2)
            j0 = jnp.minimum(jnp.maximum(kpj.astype(jnp.int32), 0), WMAP - 2)
            f_sc[sl] = val * vf
            f_kx[sl] = kpj * 16.0 + 7.5
            f_ky[sl] = kpi * 16.0 + 7.5
            f_vd[sl] = vf
            f_wi[sl] = kpi - i0.astype(jnp.float32)
            f_wj[sl] = kpj - j0.astype(jnp.float32)
            f_bs[sl] = (i0 * WMAP + j0).astype(jnp.float32)
            return 0
        lax.fori_loop(0, KTOP // 16, cb4, 0)

        for fi, fb in enumerate((f_sc, f_kx, f_ky, f_vd, f_wi, f_wj, f_bs)):
            pltpu.sync_copy(fb, sp_fields.at[pl.ds(fi * NCAND + l * KTOP, KTOP)])

    plsc.subcore_barrier()

    @pl.when(wid == 0)
    def _phase_b():
        pltpu.sync_copy(sp_fields.at[pl.ds(0, NCAND)], map_v.at[pl.ds(0, NCAND)])

        def pcb(tt, cnt):
            for u in range(4):
                t = tt * 4 + u
                v = map_v[pl.ds(t * 16, 16)]
                m = v > 0.0
                csm = plsc.cumsum(m.astype(jnp.int32))
                pos = jnp.minimum(cnt + csm - 1, NCAND - 1)
                plsc.store_scatter(skey_a, [pos],
                                   plsc.bitcast(v, jnp.int32), mask=m)
                plsc.store_scatter(sidx_a, [pos], _iota16() + t * 16, mask=m)
                cnt = cnt + jnp.sum(m.astype(jnp.int32))
            return cnt
        NP = lax.fori_loop(0, NCAND // 64, pcb, jnp.int32(0))
        NP16 = (NP + 15) // 16 * 16
        for q in range(4):
            skey_a[pl.ds(NP16 + q * 16, 16)] = jnp.zeros((16,), jnp.int32)
        skey_a[pl.ds(NP, 16)] = jnp.zeros((16,), jnp.int32)
        for pno in range(7):
            kin, iin, kout, iout = [(skey_a, sidx_a, skey_b, sidx_b),
                                    (skey_b, sidx_b, skey_a, sidx_a)][pno % 2]
            _radix_pass(kin, iin, kout, iout, hist_v, offs_v,
                        (NP + 63) // 64 * 4, pno * 5)
        def zcb(tt, cnt):
            for u in range(4):
                t = tt * 4 + u
                v = map_v[pl.ds(t * 16, 16)]
                m = v <= 0.0
                csm = plsc.cumsum(m.astype(jnp.int32))
                pos = jnp.minimum(cnt + csm - 1, NCAND - 1)
                plsc.store_scatter(skey_a, [pos], _iota16() + t * 16, mask=m)
                cnt = cnt + jnp.sum(m.astype(jnp.int32))
            return cnt
        lax.fori_loop(0, NCAND // 64, zcb, jnp.int32(0))

        def fb(i, _):
            rv = _iota16() + i * 16
            m = rv < NP
            apos = jnp.where(m, rv, 0)
            av = plsc.load_gather(sidx_b, [apos])
            bpos = jnp.where(m, 0, jnp.minimum(rv - NP, NCAND - 1))
            bv = plsc.load_gather(skey_a, [bpos])
            sidx_a[pl.ds(i * 16, 16)] = jnp.where(m, av, bv)
            return 0
        lax.fori_loop(0, MAXF // 16, fb, 0)
        pltpu.sync_copy(sidx_a.at[pl.ds(0, MAXF)], sp_sortedp)

    plsc.subcore_barrier()

    K = MAXF // NW
    base = wid * K
    pltpu.sync_copy(sp_sortedp.at[pl.ds(base, K)], spv_v)

    def _gather_field(fi, dst_loc):
        pltpu.sync_copy(sp_fields.at[pl.ds(fi * NCAND, NCAND)], map_v.at[pl.ds(0, NCAND)])

        def gb(i, _):
            p = spv_v[pl.ds(i * 16, 16)]
            dst_loc[pl.ds(i * 16, 16)] = plsc.load_gather(map_v, [p])
            return 0
        lax.fori_loop(0, K // 16, gb, 0)

    _gather_field(0, s_loc)
    pltpu.sync_copy(s_loc, scores_o.at[pl.ds(base, K)])
    _gather_field(1, w_loc)

    def kx_b(i, _):
        sl = pl.ds(i * 16, 16)
        plsc.store_scatter(kp_loc, [(_iota16() + i * 16) * 2], w_loc[sl])
        return 0
    lax.fori_loop(0, K // 16, kx_b, 0)
    _gather_field(2, w_loc)

    def ky_b(i, _):
        sl = pl.ds(i * 16, 16)
        plsc.store_scatter(kp_loc, [(_iota16() + i * 16) * 2 + 1], w_loc[sl])
        return 0
    lax.fori_loop(0, K // 16, ky_b, 0)
    pltpu.sync_copy(kp_loc, kpflat_o.at[pl.ds(base * 2, K * 2)])
    _gather_field(3, s_loc)
    pltpu.sync_copy(s_loc, msk_o.at[pl.ds(base, K)])
    for lv in range(NLVL):
        def lv_b(i, _):
            p = spv_v[pl.ds(i * 16, 16)]
            lvl_i = lax.shift_right_logical(p, 11)
            s_loc[pl.ds(i * 16, 16)] = (lvl_i == lv).astype(jnp.float32)
            return 0
        lax.fori_loop(0, K // 16, lv_b, 0)
        pltpu.sync_copy(s_loc, lvl_o.at[pl.ds(lv * MAXF + base, K)])
    _gather_field(4, w_loc)

    def wi_b(i, _):
        s_loc[pl.ds(i * 16, 16)] = w_loc[pl.ds(i * 16, 16)]
        return 0
    lax.fori_loop(0, K // 16, wi_b, 0)
    _gather_field(5, w_loc)

    def wt_b(i, _):
        sl = pl.ds(i * 16, 16)
        wif = s_loc[sl]
        wjf = w_loc[sl]
        wts4_loc[pl.ds(0 * K + i * 16, 16)] = (1.0 - wif) * (1.0 - wjf)
        wts4_loc[pl.ds(1 * K + i * 16, 16)] = (1.0 - wif) * wjf
        wts4_loc[pl.ds(2 * K + i * 16, 16)] = wif * (1.0 - wjf)
        wts4_loc[pl.ds(3 * K + i * 16, 16)] = wif * wjf
        return 0
    lax.fori_loop(0, K // 16, wt_b, 0)
    for c in range(4):
        pltpu.sync_copy(wts4_loc.at[pl.ds(c * K, K)],
                        wts_o.at[pl.ds(c * MAXF + base, K)])
    _gather_field(6, w_loc)

    def bs_b(i, _):
        b_loc[pl.ds(i * 16, 16)] = w_loc[pl.ds(i * 16, 16)].astype(jnp.int32)
        return 0
    lax.fori_loop(0, K // 16, bs_b, 0)
    for ch in range(3):
        pltpu.sync_copy(xds.at[pl.ds(ch * HW, HW)], map_v)
        for ci, off in enumerate((0, 1, WMAP, WMAP + 1)):
            def xc_b(i, _, ci=ci, off=off, ch=ch):
                b = b_loc[pl.ds(i * 16, 16)]
                g = plsc.load_gather(map_v, [b + off])
                pos = ci * (K * 3) + (_iota16() + i * 16) * 3 + ch
                plsc.store_scatter(xc4_loc, [pos], g)
                return 0
            lax.fori_loop(0, K // 16, xc_b, 0)
    for c in range(4):
        pltpu.sync_copy(xc4_loc.at[pl.ds(c * (K * 3), K * 3)],
                        xcflat_o.at[pl.ds(c * MAXF * 3 + base * 3, K * 3)])


def _select_order(snms3, dispi3, dispj3, X):
    mesh = plsc.VectorSubcoreMesh(core_axis_name="c", subcore_axis_name="s",
                                  num_cores=1)
    f32 = jnp.float32
    run = pl.kernel(
        _k2_body,
        out_type=(
            jax.ShapeDtypeStruct((MAXF,), f32),
            jax.ShapeDtypeStruct((MAXF * 2,), f32),
            jax.ShapeDtypeStruct((4 * MAXF * 3,), f32),
            jax.ShapeDtypeStruct((4 * MAXF,), f32),
            jax.ShapeDtypeStruct((NLVL * MAXF,), f32),
            jax.ShapeDtypeStruct((MAXF,), f32),
        ),
        mesh=mesh,
        compiler_params=pltpu.CompilerParams(needs_layout_passes=False),
        scratch_types=[
            pltpu.VMEM((HW,), f32),
            pltpu.VMEM((HW,), f32),
            pltpu.VMEM((SORTN,), jnp.int32),
            pltpu.VMEM((SORTN,), jnp.int32),
            pltpu.VMEM((SORTN,), jnp.int32),
            pltpu.VMEM((SORTN,), jnp.int32),
            pltpu.VMEM((KTOP + 16,), jnp.int32),
            pltpu.VMEM((512,), jnp.int32),
            pltpu.VMEM((512,), jnp.int32),
            pltpu.VMEM((KTOP,), jnp.int32),
            pltpu.VMEM((KTOP,), f32),
            pltpu.VMEM((KTOP,), f32),
            pltpu.VMEM((KTOP,), f32),
            pltpu.VMEM((KTOP,), f32),
            pltpu.VMEM((KTOP,), f32),
            pltpu.VMEM((KTOP,), f32),
            pltpu.VMEM((KTOP,), f32),
            pltpu.VMEM((KTOP,), f32),
            pltpu.VMEM((KTOP,), f32),
            pltpu.VMEM((KTOP,), f32),
            pltpu.VMEM((MAXF // NW,), jnp.int32),
            pltpu.VMEM((MAXF // NW,), jnp.int32),
            pltpu.VMEM((MAXF // NW,), f32),
            pltpu.VMEM((MAXF // NW * 2,), f32),
            pltpu.VMEM((MAXF // NW,), f32),
            pltpu.VMEM((MAXF // NW * 4,), f32),
            pltpu.VMEM((MAXF // NW * 3 * 4,), f32),
            pltpu.VMEM_SHARED((7 * NCAND,), f32),
            pltpu.VMEM_SHARED((MAXF,), jnp.int32),
        ],
    )
    return run(snms3, dispi3, dispj3, X)


def _k3_body(xc_ref, wts_ref, lvl_ref, msk_ref, w_ref, out_ref):
    acc = jnp.zeros((out_ref.shape[1], C), jnp.float32)
    for l in range(NLVL):
        Wb = w_ref[l].astype(jnp.bfloat16)
        lacc = jnp.zeros((out_ref.shape[1], C), jnp.float32)
        for c in range(4):
            A = xc_ref[c]
            Fb = jax.lax.dot_general(A.astype(jnp.bfloat16), Wb,
                                     (((1,), (0,)), ((), ())),
                                     preferred_element_type=jnp.float32)
            Fb = jnp.maximum(Fb, 0.0)
            lacc = lacc + wts_ref[c][:, None] * Fb
        acc = acc + lvl_ref[l][:, None] * lacc
    desc = acc * msk_ref[0][:, None]
    nrm = jnp.sqrt(jnp.sum(desc * desc, axis=1, keepdims=True))
    out_ref[0] = desc / (nrm + 1e-8)


def _descriptors(xc, wts, lvl1h, vmask, Wcat):
    R = 512
    return pl.pallas_call(
        _k3_body,
        grid=(MAXF // R,),
        in_specs=[
            pl.BlockSpec((4, R, 3), lambda b: (0, b, 0)),
            pl.BlockSpec((4, R), lambda b: (0, b)),
            pl.BlockSpec((NLVL, R), lambda b: (0, b)),
            pl.BlockSpec((1, R), lambda b: (0, b)),
            pl.BlockSpec((NLVL, 3, C), lambda b: (0, 0, 0)),
        ],
        out_specs=pl.BlockSpec((1, R, C), lambda b: (b, 0, 0)),
        out_shape=jax.ShapeDtypeStruct((MAXF // R, R, C), jnp.float32),
    )(xc, wts, lvl1h, vmask, Wcat)


def kernel(images, W_early, W_middle, W_deep, v_early, v_middle, v_deep):
    img = images[0]
    x_ds = _downsample(img)
    X = x_ds.reshape(3, HW)
    Wcat = jnp.stack([W_early, W_middle, W_deep], axis=0)
    vcat = jnp.stack([v_early, v_middle, v_deep], axis=0).reshape(NLVL, 1, C)

    score3 = _score_maps(X, Wcat, vcat).reshape(NLVL, HMAP, WMAP)
    s_nms, disp_i, disp_j = _nms_disp(score3)

    scores_out, kpflat, xcflat, wtsf, lvlf, mskflat = _select_order(
        s_nms.reshape(NLVL * HW), disp_i.reshape(NLVL * HW),
        disp_j.reshape(NLVL * HW), X.reshape(NLVL * HW))
    wts = wtsf.reshape(4, MAXF)
    lvl1h = lvlf.reshape(NLVL, MAXF)

    keypoints = kpflat.reshape(MAXF, 2)
    xc = xcflat.reshape(4, MAXF, 3)
    vmask = mskflat.reshape(1, MAXF)
    descriptors = _descriptors(xc, wts, lvl1h, vmask, Wcat).reshape(MAXF, C)
    return keypoints, descriptors, scores_out
